# baseline (speedup 1.0000x reference)
import functools
import math

import jax
import jax.numpy as jnp
from jax import lax
from jax.experimental import pallas as pl
from jax.experimental.pallas import tpu as pltpu
from jax.experimental.pallas import tpu_sc as plsc

BACKOFF_PROB = 1e-10
CLAMP = 20.0

_NC = 2
_NS = 16
_NW = _NC * _NS
_LANES = 16


def _sc_body(tidx_hbm, nidx_hbm, xf_hbm, w_hbm, b_hbm, n_hbm,
             ts_out, bt_out, nt_out, wn_out, bn_out, nn_out,
             tidx0_v, tidx1_v, nidx_v, wt_v, x_v, wn_v,
             bt_v, ntv_v, bn_v, nn_v, ts_v,
             s0, s1, s2, s3, s4, s5, s6, s7, s8,
             t_per_w, n_per_w, d):
    wid = lax.axis_index("s") * _NC + lax.axis_index("c")
    tb = wid * t_per_w
    nb = wid * n_per_w
    half = t_per_w // 2
    pltpu.sync_copy(tidx_hbm.at[pl.ds(tb, half)], tidx0_v)
    pltpu.sync_copy(tidx_hbm.at[pl.ds(tb + half, half)], tidx1_v)
    pltpu.sync_copy(nidx_hbm.at[pl.ds(nb, n_per_w)], nidx_v)
    cp_x0 = pltpu.async_copy(xf_hbm.at[pl.ds(tb, half)],
                             x_v.at[pl.ds(0, half)], s0)
    cp_w0 = pltpu.async_copy(w_hbm.at[tidx0_v], wt_v.at[pl.ds(0, half)], s1)
    cp_x1 = pltpu.async_copy(xf_hbm.at[pl.ds(tb + half, half)],
                             x_v.at[pl.ds(half, half)], s2)
    cp_w1 = pltpu.async_copy(w_hbm.at[tidx1_v], wt_v.at[pl.ds(half, half)], s3)
    cp_bt = pltpu.async_copy(b_hbm.at[tidx0_v], bt_v.at[pl.ds(0, half)], s4)
    cp_bt2 = pltpu.async_copy(b_hbm.at[tidx1_v], bt_v.at[pl.ds(half, half)], s4)
    cp_wn = pltpu.async_copy(w_hbm.at[nidx_v], wn_v, s5)
    cp_nt = pltpu.async_copy(n_hbm.at[tidx0_v], ntv_v.at[pl.ds(0, half)], s6)
    cp_nt2 = pltpu.async_copy(n_hbm.at[tidx1_v], ntv_v.at[pl.ds(half, half)], s6)
    cp_bn = pltpu.async_copy(b_hbm.at[nidx_v], bn_v, s7)
    cp_nn = pltpu.async_copy(n_hbm.at[nidx_v], nn_v, s8)

    n_blk = t_per_w // _LANES
    grp = 8
    zero = jnp.zeros((_LANES,), jnp.float32)
    waits = {0: (cp_x0, cp_w0), n_blk // 2: (cp_x1, cp_w1)}

    for blk in range(n_blk):
        if blk in waits:
            wx, ww = waits[blk]
            wx.wait()
            ww.wait()
        for r in range(_LANES):
            j = blk * _LANES + r

            def c_body(c8, accs, j=j):
                a0, a1, a2, a3 = accs
                for u in range(grp):
                    start = (c8 * grp + u) * _LANES
                    xa = x_v[j, pl.ds(start, _LANES)]
                    wa = wt_v[j, pl.ds(start, _LANES)]
                    p = xa * wa
                    if u % 4 == 0:
                        a0 = a0 + p
                    elif u % 4 == 1:
                        a1 = a1 + p
                    elif u % 4 == 2:
                        a2 = a2 + p
                    else:
                        a3 = a3 + p
                return (a0, a1, a2, a3)

            a0, a1, a2, a3 = lax.fori_loop(0, d // (grp * _LANES), c_body,
                                           (zero, zero, zero, zero))
            ts_v[j, :] = (a0 + a1) + (a2 + a3)

    pltpu.sync_copy(ts_v, ts_out.at[pl.ds(tb, t_per_w)])
    cp_bt.wait()
    cp_bt2.wait()
    pltpu.sync_copy(bt_v, bt_out.at[pl.ds(tb, t_per_w)])
    cp_wn.wait()
    pltpu.sync_copy(wn_v, wn_out.at[pl.ds(nb, n_per_w)])
    cp_nt.wait()
    cp_nt2.wait()
    pltpu.sync_copy(ntv_v, nt_out.at[pl.ds(tb, t_per_w)])
    cp_bn.wait()
    pltpu.sync_copy(bn_v, bn_out.at[pl.ds(nb, n_per_w)])
    cp_nn.wait()
    pltpu.sync_copy(nn_v, nn_out.at[pl.ds(nb, n_per_w)])


def _make_sc(bn_count, kpad, d):
    t_per_w = bn_count // _NW
    n_per_w = kpad // _NW
    mesh = plsc.VectorSubcoreMesh(core_axis_name="c", subcore_axis_name="s")
    return pl.kernel(
        functools.partial(_sc_body, t_per_w=t_per_w, n_per_w=n_per_w, d=d),
        mesh=mesh,
        out_type=[
            jax.ShapeDtypeStruct((bn_count, _LANES), jnp.float32),
            jax.ShapeDtypeStruct((bn_count,), jnp.float32),
            jax.ShapeDtypeStruct((bn_count,), jnp.float32),
            jax.ShapeDtypeStruct((kpad, d), jnp.float32),
            jax.ShapeDtypeStruct((kpad,), jnp.float32),
            jax.ShapeDtypeStruct((kpad,), jnp.float32),
        ],
        scratch_types=[
            pltpu.VMEM((t_per_w // 2,), jnp.int32),
            pltpu.VMEM((t_per_w // 2,), jnp.int32),
            pltpu.VMEM((n_per_w,), jnp.int32),
            pltpu.VMEM((t_per_w, d), jnp.float32),
            pltpu.VMEM((t_per_w, d), jnp.float32),
            pltpu.VMEM((n_per_w, d), jnp.float32),
            pltpu.VMEM((t_per_w,), jnp.float32),
            pltpu.VMEM((t_per_w,), jnp.float32),
            pltpu.VMEM((n_per_w,), jnp.float32),
            pltpu.VMEM((n_per_w,), jnp.float32),
            pltpu.VMEM((t_per_w, _LANES), jnp.float32),
        ] + [pltpu.SemaphoreType.DMA] * 9,
    )


def _tc_loss_body(x_ref, wn_ref, tsp_ref, bt_ref, nt_ref, bn_ref, nn_ref,
                  out_ref, *, bn_count, k, kpad, norm_term):
    x = x_ref[...]
    wn = wn_ref[...]
    ts = jnp.sum(tsp_ref[...], axis=1) + bt_ref[...]
    ns = lax.dot_general(x, wn, (((1,), (1,)), ((), ())),
                         precision=lax.Precision.HIGHEST,
                         preferred_element_type=jnp.float32)
    ns = ns + bn_ref[...][None, :]
    pm = jnp.exp(jnp.minimum(ts - norm_term, CLAMP))
    pnm = jnp.exp(jnp.minimum(ns - norm_term, CLAMP))
    kpt = k * nt_ref[...]
    kpn = k * nn_ref[...][None, :]
    p_true = pm / (pm + kpt + BACKOFF_PROB)
    p_noise = kpn / (pnm + kpn + BACKOFF_PROB)
    lp = jnp.log(p_noise + BACKOFF_PROB)
    col = lax.broadcasted_iota(jnp.int32, lp.shape, 1)
    lp = jnp.where(col < k, lp, 0.0)
    loss = -(jnp.log(p_true + BACKOFF_PROB) + jnp.sum(lp, axis=1))
    out_ref[...] = jnp.sum(loss).reshape(1, 1) / bn_count


def kernel(target, input, weight, bias, noise, noise_idx):
    b, n, d = input.shape
    bn_count = b * n
    v = weight.shape[0]
    k = noise_idx.shape[0]
    norm_term = float(math.log(v))

    kpad = ((k + 8 * _NW - 1) // (8 * _NW)) * (8 * _NW)

    tflat = target.reshape(-1).astype(jnp.int32)
    nidx = jnp.pad(noise_idx.astype(jnp.int32), (0, kpad - k))
    x2 = input.reshape(bn_count, d)

    tsp, bt, nt, wn_rows, bn_vals, nn_vals = _make_sc(bn_count, kpad, d)(
        tflat, nidx, x2, weight, bias, noise)

    out = pl.pallas_call(
        functools.partial(_tc_loss_body, bn_count=bn_count, k=k, kpad=kpad,
                          norm_term=norm_term),
        out_shape=jax.ShapeDtypeStruct((1, 1), jnp.float32),
    )(x2, wn_rows, tsp, bt, nt, bn_vals, nn_vals)
    return out[0, 0]
